# baseline (speedup 1.0000x reference)
import jax
import jax.numpy as jnp
from jax.experimental import pallas as pl


def _router_kernel(x_ref, w_ref, b_ref, probs_ref, logits_ref, z_ref):
    i = pl.program_id(0)
    logits = jax.lax.dot_general(
        x_ref[...], w_ref[...],
        dimension_numbers=(((1,), (0,)), ((), ())),
        preferred_element_type=jnp.float32,
        precision=jax.lax.Precision.HIGHEST,
    )
    logits = logits + b_ref[...]
    logits_ref[...] = logits
    m = jnp.max(logits, axis=-1, keepdims=True)
    e = jnp.exp(logits - m)
    s = jnp.sum(e, axis=-1, keepdims=True)
    probs_ref[...] = e / s
    log_z = jnp.log(s) + m
    part = jnp.sum(log_z * log_z).reshape(1, 1)

    @pl.when(i == 0)
    def _init():
        z_ref[...] = jnp.zeros((1, 1), jnp.float32)

    z_ref[...] += part


def kernel(token_inputs, W, b, num_experts, expert_capacity):
    G, T, H = token_inputs.shape
    E = W.shape[1]
    M = G * T
    x = token_inputs.reshape(M, H)
    BM = 512

    probs, logits, zsum = pl.pallas_call(
        _router_kernel,
        grid=(M // BM,),
        in_specs=[
            pl.BlockSpec((BM, H), lambda i: (i, 0)),
            pl.BlockSpec((H, E), lambda i: (0, 0)),
            pl.BlockSpec((1, E), lambda i: (0, 0)),
        ],
        out_specs=[
            pl.BlockSpec((BM, E), lambda i: (i, 0)),
            pl.BlockSpec((BM, E), lambda i: (i, 0)),
            pl.BlockSpec((1, 1), lambda i: (0, 0)),
        ],
        out_shape=[
            jax.ShapeDtypeStruct((M, E), jnp.float32),
            jax.ShapeDtypeStruct((M, E), jnp.float32),
            jax.ShapeDtypeStruct((1, 1), jnp.float32),
        ],
    )(x, W, b.reshape(1, E))

    z_loss = zsum[0, 0] / M
    return probs.reshape(G, T, E), logits.reshape(G, T, E), z_loss


# speedup vs baseline: 1.8408x; 1.8408x over previous
import jax
import jax.numpy as jnp
from jax.experimental import pallas as pl


def _router_kernel(x_ref, w_ref, b_ref, probs_ref, logits_ref, z_ref):
    i = pl.program_id(0)
    logits = jax.lax.dot_general(
        x_ref[...], w_ref[...],
        dimension_numbers=(((1,), (0,)), ((), ())),
        preferred_element_type=jnp.float32,
    )
    logits = logits + b_ref[...]
    logits_ref[...] = logits
    m = jnp.max(logits, axis=-1, keepdims=True)
    e = jnp.exp(logits - m)
    s = jnp.sum(e, axis=-1, keepdims=True)
    probs_ref[...] = e / s
    log_z = jnp.log(s) + m
    part = jnp.sum(log_z * log_z).reshape(1, 1)

    @pl.when(i == 0)
    def _init():
        z_ref[...] = jnp.zeros((1, 1), jnp.float32)

    z_ref[...] += part


def kernel(token_inputs, W, b, num_experts, expert_capacity):
    G, T, H = token_inputs.shape
    E = W.shape[1]
    M = G * T
    x = token_inputs.reshape(M, H)
    BM = 512

    probs, logits, zsum = pl.pallas_call(
        _router_kernel,
        grid=(M // BM,),
        in_specs=[
            pl.BlockSpec((BM, H), lambda i: (i, 0)),
            pl.BlockSpec((H, E), lambda i: (0, 0)),
            pl.BlockSpec((1, E), lambda i: (0, 0)),
        ],
        out_specs=[
            pl.BlockSpec((BM, E), lambda i: (i, 0)),
            pl.BlockSpec((BM, E), lambda i: (i, 0)),
            pl.BlockSpec((1, 1), lambda i: (0, 0)),
        ],
        out_shape=[
            jax.ShapeDtypeStruct((M, E), jnp.float32),
            jax.ShapeDtypeStruct((M, E), jnp.float32),
            jax.ShapeDtypeStruct((1, 1), jnp.float32),
        ],
    )(x, W, b.reshape(1, E))

    z_loss = zsum[0, 0] / M
    return probs.reshape(G, T, E), logits.reshape(G, T, E), z_loss


# speedup vs baseline: 2.0546x; 1.1161x over previous
import jax
import jax.numpy as jnp
from jax.experimental import pallas as pl


def _router_kernel(x_ref, w_ref, b_ref, probs_ref, logits_ref, z_ref):
    i = pl.program_id(0)
    logits = jax.lax.dot_general(
        x_ref[...], w_ref[...],
        dimension_numbers=(((1,), (0,)), ((), ())),
        preferred_element_type=jnp.float32,
    )
    logits = logits + b_ref[...]
    logits_ref[...] = logits
    m = jnp.max(logits, axis=-1, keepdims=True)
    e = jnp.exp(logits - m)
    s = jnp.sum(e, axis=-1, keepdims=True)
    probs_ref[...] = e / s
    log_z = jnp.log(s) + m
    part = jnp.sum(log_z * log_z).reshape(1, 1)

    @pl.when(i == 0)
    def _init():
        z_ref[...] = jnp.zeros((1, 1), jnp.float32)

    z_ref[...] += part


def kernel(token_inputs, W, b, num_experts, expert_capacity):
    G, T, H = token_inputs.shape
    E = W.shape[1]
    M = G * T
    x = token_inputs.reshape(M, H)
    BM = 1024

    probs, logits, zsum = pl.pallas_call(
        _router_kernel,
        grid=(M // BM,),
        in_specs=[
            pl.BlockSpec((BM, H), lambda i: (i, 0)),
            pl.BlockSpec((H, E), lambda i: (0, 0)),
            pl.BlockSpec((1, E), lambda i: (0, 0)),
        ],
        out_specs=[
            pl.BlockSpec((BM, E), lambda i: (i, 0)),
            pl.BlockSpec((BM, E), lambda i: (i, 0)),
            pl.BlockSpec((1, 1), lambda i: (0, 0)),
        ],
        out_shape=[
            jax.ShapeDtypeStruct((M, E), jnp.float32),
            jax.ShapeDtypeStruct((M, E), jnp.float32),
            jax.ShapeDtypeStruct((1, 1), jnp.float32),
        ],
    )(x, W, b.reshape(1, E))

    z_loss = zsum[0, 0] / M
    return probs.reshape(G, T, E), logits.reshape(G, T, E), z_loss
